# Initial kernel scaffold; baseline (speedup 1.0000x reference)
import jax
import jax.numpy as jnp
from jax.experimental import pallas as pl


def kernel(x, edge, W, a):
    raise NotImplementedError("write your pallas kernel here")


# speedup vs baseline: 55.8963x; 55.8963x over previous
import functools

import jax
import jax.numpy as jnp
from jax import lax
from jax.experimental import pallas as pl
from jax.experimental.pallas import tpu as pltpu
from jax.experimental.pallas import tpu_sc as plsc

_N = 10000
_E = 320000
_D = 128
_SLOPE = 0.2

_NT = 16
_EPT = _E // _NT
_WIN = 80
_NWIN = _EPT // _WIN
_ROWS = _E // _WIN
_RPT = _ROWS // _NT
_NZ = 10240
_ZSL = _NZ // _NT
_LANES = 16


def _st_body(x_ref, w_ref, a2_ref, st_ref):
    ut = lax.dot_general(
        a2_ref[...], w_ref[...], (((0,), (1,)), ((), ())),
        preferred_element_type=jnp.float32, precision=lax.Precision.HIGHEST)
    st_ref[...] = lax.dot_general(
        ut, x_ref[...], (((1,), (1,)), ((), ())),
        preferred_element_type=jnp.float32, precision=lax.Precision.HIGHEST)


_st_call = pl.pallas_call(
    _st_body, out_shape=jax.ShapeDtypeStruct((2, _N), jnp.float32))


def _edge_body(s_hbm, t_hbm, src_hbm, dst_hbm, srcw_hbm, out_hbm,
               src_v, dst_v, srcw_v, s_v, t_v, p_v, z_v, zero_v, z_sh):
    tid = lax.axis_index("s")
    ebase = tid * _EPT

    def zbody(j, c):
        zero_v[pl.ds(j * _LANES, _LANES)] = jnp.zeros((_LANES,), jnp.float32)
        return c
    lax.fori_loop(0, _ZSL // _LANES, zbody, 0)
    pltpu.sync_copy(zero_v, z_sh.at[pl.ds(tid * _ZSL, _ZSL)])

    pltpu.sync_copy(src_hbm.at[pl.ds(ebase, _EPT)], src_v)
    pltpu.sync_copy(dst_hbm.at[pl.ds(ebase, _EPT)], dst_v)
    pltpu.sync_copy(srcw_hbm.at[tid], srcw_v)
    pltpu.sync_copy(s_hbm, s_v)
    pltpu.sync_copy(t_hbm, t_v)
    plsc.subcore_barrier()

    def pbody(i, c):
        sl = pl.ds(i * _LANES, _LANES)
        e = plsc.load_gather(s_v, [src_v[sl]]) + plsc.load_gather(t_v, [dst_v[sl]])
        e = jnp.where(e >= 0.0, e, e * _SLOPE)
        p_v[sl] = jnp.exp(e)
        return c
    lax.fori_loop(0, _EPT // _LANES, pbody, 0)

    def wbody(w, c):
        pltpu.sync_copy(p_v.at[pl.ds(w * _WIN, _WIN)],
                        z_sh.at[srcw_v.at[w]], add=True)
        return c
    lax.fori_loop(0, _NWIN, wbody, 0)
    plsc.subcore_barrier()

    pltpu.sync_copy(z_sh.at[pl.ds(0, _N)], z_v)

    def obody(i, c):
        sl = pl.ds(i * _LANES, _LANES)
        zg = plsc.load_gather(z_v, [src_v[sl]])
        p_v[sl] = p_v[sl] / (zg + 1e-16)
        return c
    lax.fori_loop(0, _EPT // _LANES, obody, 0)
    pltpu.sync_copy(p_v, out_hbm.at[pl.ds(ebase, _EPT)])


_edge_call = pl.kernel(
    _edge_body,
    out_type=jax.ShapeDtypeStruct((_E,), jnp.float32),
    mesh=plsc.VectorSubcoreMesh(core_axis_name="c", subcore_axis_name="s"),
    compiler_params=pltpu.CompilerParams(needs_layout_passes=False),
    scratch_types=[
        pltpu.VMEM((_EPT,), jnp.int32),
        pltpu.VMEM((_EPT,), jnp.int32),
        pltpu.VMEM((_RPT, _WIN), jnp.int32),
        pltpu.VMEM((_N,), jnp.float32),
        pltpu.VMEM((_N,), jnp.float32),
        pltpu.VMEM((_EPT,), jnp.float32),
        pltpu.VMEM((_N,), jnp.float32),
        pltpu.VMEM((_ZSL,), jnp.float32),
        pltpu.VMEM_SHARED((_NZ,), jnp.float32),
    ],
)


def kernel(x, edge, W, a):
    src = edge[0].astype(jnp.int32)
    dst = edge[1].astype(jnp.int32)
    a256 = a.reshape(2 * _D)
    a2 = jnp.stack([a256[:_D], a256[_D:]], axis=1)
    st2 = _st_call(x, W, a2)
    srcw = src.reshape(_NT, _RPT, _WIN)
    att = _edge_call(st2[0], st2[1], src, dst, srcw)
    return att.reshape(_E, 1)


# speedup vs baseline: 66.6173x; 1.1918x over previous
import functools

import jax
import jax.numpy as jnp
from jax import lax
from jax.experimental import pallas as pl
from jax.experimental.pallas import tpu as pltpu
from jax.experimental.pallas import tpu_sc as plsc

_N = 10000
_E = 320000
_D = 128
_SLOPE = 0.2

_NT = 16
_EPT = _E // _NT
_WIN = 80
_NWIN = _EPT // _WIN
_ROWS = _E // _WIN
_RPT = _ROWS // _NT
_NZ = 10240
_ZSL = _NZ // _NT
_LANES = 16


def _st_body(x_ref, w_ref, a2_ref, st_ref):
    ut = lax.dot_general(
        a2_ref[...], w_ref[...], (((0,), (1,)), ((), ())),
        preferred_element_type=jnp.float32, precision=lax.Precision.HIGHEST)
    st_ref[...] = lax.dot_general(
        ut, x_ref[...], (((1,), (1,)), ((), ())),
        preferred_element_type=jnp.float32, precision=lax.Precision.HIGHEST)


_st_call = pl.pallas_call(
    _st_body, out_shape=jax.ShapeDtypeStruct((2, _N), jnp.float32))


def _edge_body(s_hbm, t_hbm, src_hbm, dst_hbm, srcw_hbm, out_hbm,
               src_v, dst_v, srcw_v, s_v, t_v, p_v, z_v, zero_v, z_sh, sem):
    tid = lax.axis_index("s")
    ebase = tid * _EPT

    def zbody(j, c):
        zero_v[pl.ds(j * _LANES, _LANES)] = jnp.zeros((_LANES,), jnp.float32)
        return c
    lax.fori_loop(0, _ZSL // _LANES, zbody, 0)
    pltpu.sync_copy(zero_v, z_sh.at[pl.ds(tid * _ZSL, _ZSL)])

    pltpu.sync_copy(src_hbm.at[pl.ds(ebase, _EPT)], src_v)
    pltpu.sync_copy(dst_hbm.at[pl.ds(ebase, _EPT)], dst_v)
    pltpu.sync_copy(srcw_hbm.at[tid], srcw_v)
    pltpu.sync_copy(s_hbm, s_v)
    pltpu.sync_copy(t_hbm, t_v)
    plsc.subcore_barrier()

    def wbody(w, c):
        for j in range(_WIN // _LANES):
            sl = pl.ds(w * _WIN + j * _LANES, _LANES)
            e = (plsc.load_gather(s_v, [src_v[sl]])
                 + plsc.load_gather(t_v, [dst_v[sl]]))
            e = jnp.where(e >= 0.0, e, e * _SLOPE)
            p_v[sl] = jnp.exp(e)
        pltpu.async_copy(p_v.at[pl.ds(w * _WIN, _WIN)],
                         z_sh.at[srcw_v.at[w]], sem, add=True)
        return c
    lax.fori_loop(0, _NWIN, wbody, 0)

    def dbody(w, c):
        pltpu.make_async_copy(p_v.at[pl.ds(0, _WIN)],
                              z_sh.at[srcw_v.at[0]], sem).wait()
        return c
    lax.fori_loop(0, _NWIN, dbody, 0)
    plsc.subcore_barrier()

    pltpu.sync_copy(z_sh.at[pl.ds(0, _N)], z_v)

    def obody(i, c):
        sl = pl.ds(i * _LANES, _LANES)
        zg = plsc.load_gather(z_v, [src_v[sl]])
        p_v[sl] = p_v[sl] / (zg + 1e-16)
        return c
    lax.fori_loop(0, _EPT // _LANES, obody, 0)
    pltpu.sync_copy(p_v, out_hbm.at[pl.ds(ebase, _EPT)])


_edge_call = pl.kernel(
    _edge_body,
    out_type=jax.ShapeDtypeStruct((_E,), jnp.float32),
    mesh=plsc.VectorSubcoreMesh(core_axis_name="c", subcore_axis_name="s"),
    compiler_params=pltpu.CompilerParams(needs_layout_passes=False),
    scratch_types=[
        pltpu.VMEM((_EPT,), jnp.int32),
        pltpu.VMEM((_EPT,), jnp.int32),
        pltpu.VMEM((_RPT, _WIN), jnp.int32),
        pltpu.VMEM((_N,), jnp.float32),
        pltpu.VMEM((_N,), jnp.float32),
        pltpu.VMEM((_EPT,), jnp.float32),
        pltpu.VMEM((_N,), jnp.float32),
        pltpu.VMEM((_ZSL,), jnp.float32),
        pltpu.VMEM_SHARED((_NZ,), jnp.float32),
        pltpu.SemaphoreType.DMA,
    ],
)


def kernel(x, edge, W, a):
    src = edge[0].astype(jnp.int32)
    dst = edge[1].astype(jnp.int32)
    a256 = a.reshape(2 * _D)
    a2 = jnp.stack([a256[:_D], a256[_D:]], axis=1)
    st2 = _st_call(x, W, a2)
    srcw = src.reshape(_NT, _RPT, _WIN)
    att = _edge_call(st2[0], st2[1], src, dst, srcw)
    return att.reshape(_E, 1)


# speedup vs baseline: 69.8997x; 1.0493x over previous
import functools

import jax
import jax.numpy as jnp
from jax import lax
from jax.experimental import pallas as pl
from jax.experimental.pallas import tpu as pltpu
from jax.experimental.pallas import tpu_sc as plsc

_N = 10000
_E = 320000
_D = 128
_SLOPE = 0.2

_NT = 16
_EPT = _E // _NT
_WIN = 80
_NWIN = _EPT // _WIN
_ROWS = _E // _WIN
_RPT = _ROWS // _NT
_NZ = 10240
_ZSL = _NZ // _NT
_LANES = 16


def _st_body(x_ref, w_ref, a2_ref, st_ref):
    ut = lax.dot_general(
        a2_ref[...], w_ref[...], (((0,), (1,)), ((), ())),
        preferred_element_type=jnp.float32, precision=lax.Precision.HIGHEST)
    st_ref[...] = lax.dot_general(
        ut, x_ref[...], (((1,), (1,)), ((), ())),
        preferred_element_type=jnp.float32, precision=lax.Precision.HIGHEST)


_st_call = pl.pallas_call(
    _st_body, out_shape=jax.ShapeDtypeStruct((2, _N), jnp.float32))


def _edge_body(s_hbm, t_hbm, src_hbm, dst_hbm, srcw_hbm, out_hbm,
               src_v, dst_v, srcw_v, s_v, t_v, p_v, z_v, zero_v, z_sh, sem):
    tid = lax.axis_index("s")
    ebase = tid * _EPT

    def zbody(j, c):
        zero_v[pl.ds(j * _LANES, _LANES)] = jnp.zeros((_LANES,), jnp.float32)
        return c
    lax.fori_loop(0, _ZSL // _LANES, zbody, 0)
    pltpu.sync_copy(zero_v, z_sh.at[pl.ds(tid * _ZSL, _ZSL)])

    c1 = pltpu.async_copy(src_hbm.at[pl.ds(ebase, _EPT)], src_v, sem)
    c2 = pltpu.async_copy(dst_hbm.at[pl.ds(ebase, _EPT)], dst_v, sem)
    c3 = pltpu.async_copy(srcw_hbm.at[tid], srcw_v, sem)
    c4 = pltpu.async_copy(s_hbm, s_v, sem)
    c5 = pltpu.async_copy(t_hbm, t_v, sem)
    c1.wait(); c2.wait(); c3.wait(); c4.wait(); c5.wait()
    plsc.subcore_barrier()

    def wbody(w, c):
        for j in range(_WIN // _LANES):
            sl = pl.ds(w * _WIN + j * _LANES, _LANES)
            e = (plsc.load_gather(s_v, [src_v[sl]])
                 + plsc.load_gather(t_v, [dst_v[sl]]))
            e = jnp.where(e >= 0.0, e, e * _SLOPE)
            p_v[sl] = jnp.exp(e)
        pltpu.async_copy(p_v.at[pl.ds(w * _WIN, _WIN)],
                         z_sh.at[srcw_v.at[w]], sem, add=True)
        return c
    lax.fori_loop(0, _NWIN, wbody, 0)

    def dbody(w, c):
        pltpu.make_async_copy(p_v.at[pl.ds(0, _WIN)],
                              z_sh.at[srcw_v.at[0]], sem).wait()
        return c
    lax.fori_loop(0, _NWIN, dbody, 0)
    plsc.subcore_barrier()

    pltpu.sync_copy(z_sh.at[pl.ds(0, _N)], z_v)

    def rbody(i, c):
        sl = pl.ds(i * _LANES, _LANES)
        z_v[sl] = 1.0 / (z_v[sl] + 1e-16)
        return c
    lax.fori_loop(0, _N // _LANES, rbody, 0)

    def obody(w, c):
        for j in range(_WIN // _LANES):
            sl = pl.ds(w * _WIN + j * _LANES, _LANES)
            zg = plsc.load_gather(z_v, [src_v[sl]])
            p_v[sl] = p_v[sl] * zg
        return c
    lax.fori_loop(0, _NWIN, obody, 0)
    pltpu.sync_copy(p_v, out_hbm.at[pl.ds(ebase, _EPT)])


_edge_call = pl.kernel(
    _edge_body,
    out_type=jax.ShapeDtypeStruct((_E,), jnp.float32),
    mesh=plsc.VectorSubcoreMesh(core_axis_name="c", subcore_axis_name="s"),
    compiler_params=pltpu.CompilerParams(needs_layout_passes=False),
    scratch_types=[
        pltpu.VMEM((_EPT,), jnp.int32),
        pltpu.VMEM((_EPT,), jnp.int32),
        pltpu.VMEM((_RPT, _WIN), jnp.int32),
        pltpu.VMEM((_N,), jnp.float32),
        pltpu.VMEM((_N,), jnp.float32),
        pltpu.VMEM((_EPT,), jnp.float32),
        pltpu.VMEM((_N,), jnp.float32),
        pltpu.VMEM((_ZSL,), jnp.float32),
        pltpu.VMEM_SHARED((_NZ,), jnp.float32),
        pltpu.SemaphoreType.DMA,
    ],
)


def kernel(x, edge, W, a):
    src = edge[0].astype(jnp.int32)
    dst = edge[1].astype(jnp.int32)
    a256 = a.reshape(2 * _D)
    a2 = jnp.stack([a256[:_D], a256[_D:]], axis=1)
    st2 = _st_call(x, W, a2)
    srcw = src.reshape(_NT, _RPT, _WIN)
    att = _edge_call(st2[0], st2[1], src, dst, srcw)
    return att.reshape(_E, 1)
